# Initial kernel scaffold; baseline (speedup 1.0000x reference)
import jax
import jax.numpy as jnp
from jax.experimental import pallas as pl


def kernel(x, edge_index, W1, b1, W2, b2):
    raise NotImplementedError("write your pallas kernel here")


# speedup vs baseline: 2.6289x; 2.6289x over previous
import functools

import numpy as np

import jax
import jax.numpy as jnp
from jax import lax
from jax.experimental import pallas as pl
from jax.experimental.pallas import tpu as pltpu
from jax.experimental.pallas import tpu_sc as plsc

_i0 = np.int32(0)
NC = 2
NS = 16
L = 16
NW = NC * NS

SUB = 80
NSUB = 4
CPT = SUB * NSUB
EP = CPT * NW
NP = 2048


def _phase_a(g0_h, g1_h, s1_h, x_h,
             lgx_h, ps0_h, ps1_h, pc0_h, pc1_h,
             ig0, ig1, sm, is1, a, b, ones, zbuf,
             msum, mcnt, sem0, sem1):
    c = lax.axis_index("c")
    s = lax.axis_index("s")
    wid = c * NS + s
    D = x_h.shape[1]
    z16 = jnp.zeros((L,), jnp.float32)
    one16 = jnp.ones((L,), jnp.float32)

    def zrow(i, carry):
        for k in range(D // L):
            zbuf[i, pl.ds(k * L, L)] = z16
        return carry
    lax.fori_loop(0, zbuf.shape[0], zrow, jnp.int32(0))

    def orow(i, carry):
        for k in range(D // L):
            ones[i, pl.ds(k * L, L)] = one16
        return carry
    lax.fori_loop(0, SUB, orow, jnp.int32(0))

    base_r = s * (NP // NS)
    pltpu.sync_copy(zbuf, msum.at[pl.ds(base_r, 64)])
    pltpu.sync_copy(zbuf, msum.at[pl.ds(base_r + 64, 64)])
    pltpu.sync_copy(zbuf, mcnt.at[pl.ds(base_r, 64)])
    pltpu.sync_copy(zbuf, mcnt.at[pl.ds(base_r + 64, 64)])
    plsc.subcore_barrier()

    rb = wid * NSUB
    pltpu.sync_copy(g0_h.at[pl.ds(rb, NSUB)], ig0)
    pltpu.sync_copy(g1_h.at[pl.ds(rb, NSUB)], ig1)
    pltpu.sync_copy(s1_h.at[pl.ds(rb, NSUB)], is1)

    for j in range(NSUB):
        for k in range(SUB // L):
            i0 = ig0[j, pl.ds(k * L, L)]
            i1 = ig1[j, pl.ds(k * L, L)]
            sm[j, pl.ds(k * L, L)] = jnp.where(i0 == i1, NP - 8, i1)

    for j in range(NSUB):
        eb = wid * CPT + j * SUB
        ca = pltpu.async_copy(x_h.at[ig0.at[jnp.int32(j)]], a, sem0)
        cb = pltpu.async_copy(x_h.at[ig1.at[jnp.int32(j)]], b, sem1)
        ca.wait()
        cb.wait()

        def row_fn(r, carry):
            for k in range(D // L):
                va = a[r, pl.ds(k * L, L)]
                vb = b[r, pl.ds(k * L, L)]
                a[r, pl.ds(k * L, L)] = jnp.maximum(0.5 * va + 1.5 * vb, 0.0)
                b[r, pl.ds(k * L, L)] = 0.5 * va + 0.5 * vb
            return carry
        lax.fori_loop(0, SUB, row_fn, jnp.int32(0))

        pltpu.sync_copy(a, msum.at[sm.at[jnp.int32(j)]], add=True)
        pltpu.sync_copy(ones, mcnt.at[is1.at[jnp.int32(j)]], add=True)
        pltpu.sync_copy(b, lgx_h.at[pl.ds(eb, SUB)])

    plsc.subcore_barrier()

    @pl.when(c == 0)
    def _():
        pltpu.sync_copy(msum.at[pl.ds(base_r, 128)], ps0_h.at[pl.ds(base_r, 128)])
        pltpu.sync_copy(mcnt.at[pl.ds(base_r, 128)], pc0_h.at[pl.ds(base_r, 128)])

    @pl.when(c == 1)
    def _():
        pltpu.sync_copy(msum.at[pl.ds(base_r, 128)], ps1_h.at[pl.ds(base_r, 128)])
        pltpu.sync_copy(mcnt.at[pl.ds(base_r, 128)], pc1_h.at[pl.ds(base_r, 128)])


def _phase_b(lgx_h, ps0_h, ps1_h, g0_h, h_h,
             ig0, lbuf, p0, p1, sem0, sem1, sem2):
    c = lax.axis_index("c")
    s = lax.axis_index("s")
    wid = c * NS + s
    D = lbuf.shape[1]
    rb = wid * NSUB
    pltpu.sync_copy(g0_h.at[pl.ds(rb, NSUB)], ig0)
    for j in range(NSUB):
        eb = wid * CPT + j * SUB
        cl = pltpu.async_copy(lgx_h.at[pl.ds(eb, SUB)], lbuf, sem2)
        c0 = pltpu.async_copy(ps0_h.at[ig0.at[jnp.int32(j)]], p0, sem0)
        c1 = pltpu.async_copy(ps1_h.at[ig0.at[jnp.int32(j)]], p1, sem1)
        cl.wait()
        c0.wait()
        c1.wait()

        def row_fn(r, carry):
            for k in range(D // L):
                lbuf[r, pl.ds(k * L, L)] = (lbuf[r, pl.ds(k * L, L)]
                                            + p0[r, pl.ds(k * L, L)]
                                            + p1[r, pl.ds(k * L, L)])
            return carry
        lax.fori_loop(0, SUB, row_fn, jnp.int32(0))
        pltpu.sync_copy(lbuf, h_h.at[pl.ds(eb, SUB)])


def _phase_d(lgx2_h, s1_h, pb0_h, pb1_h,
             is1, v, zbuf, shared, sem0):
    c = lax.axis_index("c")
    s = lax.axis_index("s")
    wid = c * NS + s
    D = v.shape[1]
    z16 = jnp.zeros((L,), jnp.float32)

    def zrow(i, carry):
        for k in range(D // L):
            zbuf[i, pl.ds(k * L, L)] = z16
        return carry
    lax.fori_loop(0, zbuf.shape[0], zrow, jnp.int32(0))

    base_r = s * (NP // NS)
    pltpu.sync_copy(zbuf, shared.at[pl.ds(base_r, 64)])
    pltpu.sync_copy(zbuf, shared.at[pl.ds(base_r + 64, 64)])
    plsc.subcore_barrier()

    rb = wid * NSUB
    pltpu.sync_copy(s1_h.at[pl.ds(rb, NSUB)], is1)
    for j in range(NSUB):
        eb = wid * CPT + j * SUB
        pltpu.sync_copy(lgx2_h.at[pl.ds(eb, SUB)], v)
        pltpu.sync_copy(v, shared.at[is1.at[jnp.int32(j)]], add=True)

    plsc.subcore_barrier()

    @pl.when(c == 0)
    def _():
        pltpu.sync_copy(shared.at[pl.ds(base_r, 128)], pb0_h.at[pl.ds(base_r, 128)])

    @pl.when(c == 1)
    def _():
        pltpu.sync_copy(shared.at[pl.ds(base_r, 128)], pb1_h.at[pl.ds(base_r, 128)])


def _mlp_body(h_ref, w1_ref, b1_ref, w2_ref, b2_ref, o_ref):
    t = jnp.dot(h_ref[...], w1_ref[...],
                preferred_element_type=jnp.float32,
                precision=lax.Precision.HIGHEST) + b1_ref[...]
    t = jnp.maximum(t, 0.0)
    o_ref[...] = jnp.dot(t, w2_ref[...],
                         preferred_element_type=jnp.float32,
                         precision=lax.Precision.HIGHEST) + b2_ref[...]


def _fin_body(x_ref, s0_ref, s1_ref, c0_ref, c1_ref, o_ref):
    cnt = c0_ref[...][:, 0:1] + c1_ref[...][:, 0:1]
    ssum = s0_ref[...] + s1_ref[...]
    g = ssum / jnp.maximum(cnt, 1.0)
    o_ref[...] = x_ref[...] + jnp.maximum(g, 0.0)


def kernel(x, edge_index, W1, b1, W2, b2):
    N, D = x.shape
    E = edge_index.shape[1]
    pad = EP - E
    trash = NP - 8

    ei = edge_index.astype(jnp.int32)
    col0, col1 = ei[0], ei[1]
    g0 = jnp.concatenate([col0, jnp.zeros((pad,), jnp.int32)]).reshape(EP // SUB, SUB)
    g1 = jnp.concatenate([col1, jnp.zeros((pad,), jnp.int32)]).reshape(EP // SUB, SUB)
    s1 = jnp.concatenate([col1, jnp.full((pad,), trash, jnp.int32)]).reshape(EP // SUB, SUB)
    b1r = b1.reshape(1, D)
    b2r = b2.reshape(1, D)

    mesh = plsc.VectorSubcoreMesh(core_axis_name="c", subcore_axis_name="s")
    f32 = jnp.float32

    phase_a = functools.partial(
        pl.kernel, mesh=mesh,
        out_type=(jax.ShapeDtypeStruct((EP, D), f32),
                  jax.ShapeDtypeStruct((NP, D), f32),
                  jax.ShapeDtypeStruct((NP, D), f32),
                  jax.ShapeDtypeStruct((NP, D), f32),
                  jax.ShapeDtypeStruct((NP, D), f32)),
        scratch_types=[pltpu.VMEM((NSUB, SUB), jnp.int32),
                       pltpu.VMEM((NSUB, SUB), jnp.int32),
                       pltpu.VMEM((NSUB, SUB), jnp.int32),
                       pltpu.VMEM((NSUB, SUB), jnp.int32),
                       pltpu.VMEM((SUB, D), f32),
                       pltpu.VMEM((SUB, D), f32),
                       pltpu.VMEM((SUB, D), f32),
                       pltpu.VMEM((64, D), f32),
                       pltpu.VMEM_SHARED((NP, D), f32),
                       pltpu.VMEM_SHARED((NP, D), f32),
                       pltpu.SemaphoreType.DMA,
                       pltpu.SemaphoreType.DMA],
    )(_phase_a)
    lgx, ps0, ps1, pc0, pc1 = phase_a(g0, g1, s1, x)

    phase_b = functools.partial(
        pl.kernel, mesh=mesh,
        out_type=jax.ShapeDtypeStruct((EP, D), f32),
        scratch_types=[pltpu.VMEM((NSUB, SUB), jnp.int32),
                       pltpu.VMEM((SUB, D), f32),
                       pltpu.VMEM((SUB, D), f32),
                       pltpu.VMEM((SUB, D), f32),
                       pltpu.SemaphoreType.DMA,
                       pltpu.SemaphoreType.DMA,
                       pltpu.SemaphoreType.DMA],
    )(_phase_b)
    h = phase_b(lgx, ps0, ps1, g0)

    BLK = 512
    lgx2 = pl.pallas_call(
        _mlp_body,
        grid=(EP // BLK,),
        in_specs=[pl.BlockSpec((BLK, D), lambda i: (i, _i0)),
                  pl.BlockSpec((D, D), lambda i: (_i0, _i0)),
                  pl.BlockSpec((1, D), lambda i: (_i0, _i0)),
                  pl.BlockSpec((D, D), lambda i: (_i0, _i0)),
                  pl.BlockSpec((1, D), lambda i: (_i0, _i0))],
        out_specs=pl.BlockSpec((BLK, D), lambda i: (i, _i0)),
        out_shape=jax.ShapeDtypeStruct((EP, D), f32),
    )(h, W1, b1r, W2, b2r)

    phase_d = functools.partial(
        pl.kernel, mesh=mesh,
        out_type=(jax.ShapeDtypeStruct((NP, D), f32),
                  jax.ShapeDtypeStruct((NP, D), f32)),
        scratch_types=[pltpu.VMEM((NSUB, SUB), jnp.int32),
                       pltpu.VMEM((SUB, D), f32),
                       pltpu.VMEM((64, D), f32),
                       pltpu.VMEM_SHARED((NP, D), f32),
                       pltpu.SemaphoreType.DMA],
    )(_phase_d)
    pb0, pb1 = phase_d(lgx2, s1)

    RB = 400
    out = pl.pallas_call(
        _fin_body,
        grid=(N // RB,),
        in_specs=[pl.BlockSpec((RB, D), lambda i: (i, _i0)),
                  pl.BlockSpec((RB, D), lambda i: (i, _i0)),
                  pl.BlockSpec((RB, D), lambda i: (i, _i0)),
                  pl.BlockSpec((RB, D), lambda i: (i, _i0)),
                  pl.BlockSpec((RB, D), lambda i: (i, _i0))],
        out_specs=pl.BlockSpec((RB, D), lambda i: (i, _i0)),
        out_shape=jax.ShapeDtypeStruct((N, D), f32),
    )(x, pb0, pb1, pc0, pc1)
    return out


# speedup vs baseline: 3.0893x; 1.1752x over previous
import functools

import numpy as np

import jax
import jax.numpy as jnp
from jax import lax
from jax.experimental import pallas as pl
from jax.experimental.pallas import tpu as pltpu
from jax.experimental.pallas import tpu_sc as plsc

_i0 = np.int32(0)
NC = 2
NS = 16
L = 16
NW = NC * NS

SUB = 80
NSUB = 4
CPT = SUB * NSUB
EP = CPT * NW
NP = 2048


def _phase_a(g0_h, g1_h, s1_h, x_h,
             lgx_h, ps0_h, ps1_h, pc0_h, pc1_h,
             ig0, ig1, sm, is1, a0, b0, a1, b1, ones, zbuf,
             msum, mcnt,
             ga0, gb0, ga1, gb1, om0, oc0, ol0, om1, oc1, ol1):
    c = lax.axis_index("c")
    s = lax.axis_index("s")
    wid = c * NS + s
    D = x_h.shape[1]
    z16 = jnp.zeros((L,), jnp.float32)
    one16 = jnp.ones((L,), jnp.float32)

    def zrow(i, carry):
        for k in range(D // L):
            zbuf[i, pl.ds(k * L, L)] = z16
        return carry
    lax.fori_loop(0, zbuf.shape[0], zrow, jnp.int32(0))

    def orow(i, carry):
        for k in range(D // L):
            ones[i, pl.ds(k * L, L)] = one16
        return carry
    lax.fori_loop(0, SUB, orow, jnp.int32(0))

    base_r = s * (NP // NS)
    pltpu.sync_copy(zbuf, msum.at[pl.ds(base_r, 64)])
    pltpu.sync_copy(zbuf, msum.at[pl.ds(base_r + 64, 64)])
    pltpu.sync_copy(zbuf, mcnt.at[pl.ds(base_r, 64)])
    pltpu.sync_copy(zbuf, mcnt.at[pl.ds(base_r + 64, 64)])
    plsc.subcore_barrier()

    rb = wid * NSUB
    pltpu.sync_copy(g0_h.at[pl.ds(rb, NSUB)], ig0)
    pltpu.sync_copy(g1_h.at[pl.ds(rb, NSUB)], ig1)
    pltpu.sync_copy(s1_h.at[pl.ds(rb, NSUB)], is1)

    for j in range(NSUB):
        for k in range(SUB // L):
            i0 = ig0[j, pl.ds(k * L, L)]
            i1 = ig1[j, pl.ds(k * L, L)]
            sm[j, pl.ds(k * L, L)] = jnp.where(i0 == i1, NP - 8, i1)

    ab = (a0, a1)
    bb = (b0, b1)
    gsa = (ga0, ga1)
    gsb = (gb0, gb1)
    oms = (om0, om1)
    ocs = (oc0, oc1)
    ols = (ol0, ol1)

    def gather(j):
        p = j & 1
        ha = pltpu.async_copy(x_h.at[ig0.at[jnp.int32(j)]], ab[p], gsa[p])
        hb = pltpu.async_copy(x_h.at[ig1.at[jnp.int32(j)]], bb[p], gsb[p])
        return ha, hb

    pend_g = {0: gather(0)}
    pend_o = {}
    for j in range(NSUB):
        p = j & 1
        eb = wid * CPT + j * SUB
        if j + 1 < NSUB:
            if (j - 1) in pend_o:
                for hh in pend_o.pop(j - 1):
                    hh.wait()
            pend_g[j + 1] = gather(j + 1)
        ha, hb = pend_g.pop(j)
        ha.wait()
        hb.wait()

        a = ab[p]
        b = bb[p]

        def row_fn(r, carry):
            for k in range(D // L):
                va = a[r, pl.ds(k * L, L)]
                vb = b[r, pl.ds(k * L, L)]
                a[r, pl.ds(k * L, L)] = jnp.maximum(0.5 * va + 1.5 * vb, 0.0)
                b[r, pl.ds(k * L, L)] = 0.5 * va + 0.5 * vb
            return carry
        lax.fori_loop(0, SUB, row_fn, jnp.int32(0))

        hm = pltpu.async_copy(a, msum.at[sm.at[jnp.int32(j)]], oms[p], add=True)
        hc = pltpu.async_copy(ones, mcnt.at[is1.at[jnp.int32(j)]], ocs[p], add=True)
        hl = pltpu.async_copy(b, lgx_h.at[pl.ds(eb, SUB)], ols[p])
        pend_o[j] = (hm, hc, hl)

    for j in sorted(pend_o):
        for hh in pend_o[j]:
            hh.wait()
    plsc.subcore_barrier()

    @pl.when(c == 0)
    def _():
        pltpu.sync_copy(msum.at[pl.ds(base_r, 128)], ps0_h.at[pl.ds(base_r, 128)])
        pltpu.sync_copy(mcnt.at[pl.ds(base_r, 128)], pc0_h.at[pl.ds(base_r, 128)])

    @pl.when(c == 1)
    def _():
        pltpu.sync_copy(msum.at[pl.ds(base_r, 128)], ps1_h.at[pl.ds(base_r, 128)])
        pltpu.sync_copy(mcnt.at[pl.ds(base_r, 128)], pc1_h.at[pl.ds(base_r, 128)])


def _phase_b(lgx_h, ps0_h, ps1_h, g0_h, h_h,
             ig0, l0, p00, p10, l1, p01, p11,
             gl0, g00, g10, gl1, g01, g11, oh0, oh1):
    c = lax.axis_index("c")
    s = lax.axis_index("s")
    wid = c * NS + s
    D = l0.shape[1]
    rb = wid * NSUB
    pltpu.sync_copy(g0_h.at[pl.ds(rb, NSUB)], ig0)

    lb = (l0, l1)
    p0b = (p00, p01)
    p1b = (p10, p11)
    gls = (gl0, gl1)
    g0s = (g00, g01)
    g1s = (g10, g11)
    ohs = (oh0, oh1)

    def issue(j):
        p = j & 1
        eb = wid * CPT + j * SUB
        hl = pltpu.async_copy(lgx_h.at[pl.ds(eb, SUB)], lb[p], gls[p])
        h0 = pltpu.async_copy(ps0_h.at[ig0.at[jnp.int32(j)]], p0b[p], g0s[p])
        h1 = pltpu.async_copy(ps1_h.at[ig0.at[jnp.int32(j)]], p1b[p], g1s[p])
        return hl, h0, h1

    pend_g = {0: issue(0)}
    pend_o = {}
    for j in range(NSUB):
        p = j & 1
        eb = wid * CPT + j * SUB
        if j + 1 < NSUB:
            if (j - 1) in pend_o:
                pend_o.pop(j - 1).wait()
            pend_g[j + 1] = issue(j + 1)
        hl, h0, h1 = pend_g.pop(j)
        hl.wait()
        h0.wait()
        h1.wait()

        lbuf = lb[p]
        p0 = p0b[p]
        p1 = p1b[p]

        def row_fn(r, carry):
            for k in range(D // L):
                lbuf[r, pl.ds(k * L, L)] = (lbuf[r, pl.ds(k * L, L)]
                                            + p0[r, pl.ds(k * L, L)]
                                            + p1[r, pl.ds(k * L, L)])
            return carry
        lax.fori_loop(0, SUB, row_fn, jnp.int32(0))
        pend_o[j] = pltpu.async_copy(lbuf, h_h.at[pl.ds(eb, SUB)], ohs[p])

    for j in sorted(pend_o):
        pend_o[j].wait()


def _phase_d(lgx2_h, s1_h, pb0_h, pb1_h,
             is1, v0, v1, v2, v3, zbuf, shared,
             gv0, gv1, gv2, gv3, os0, os1):
    c = lax.axis_index("c")
    s = lax.axis_index("s")
    wid = c * NS + s
    D = v0.shape[1]
    z16 = jnp.zeros((L,), jnp.float32)

    def zrow(i, carry):
        for k in range(D // L):
            zbuf[i, pl.ds(k * L, L)] = z16
        return carry
    lax.fori_loop(0, zbuf.shape[0], zrow, jnp.int32(0))

    base_r = s * (NP // NS)
    pltpu.sync_copy(zbuf, shared.at[pl.ds(base_r, 64)])
    pltpu.sync_copy(zbuf, shared.at[pl.ds(base_r + 64, 64)])

    rb = wid * NSUB
    pltpu.sync_copy(s1_h.at[pl.ds(rb, NSUB)], is1)
    vb = (v0, v1, v2, v3)
    gvs = (gv0, gv1, gv2, gv3)
    oss = (os0, os1)
    reads = []
    for j in range(NSUB):
        eb = wid * CPT + j * SUB
        reads.append(pltpu.async_copy(lgx2_h.at[pl.ds(eb, SUB)], vb[j], gvs[j]))
    plsc.subcore_barrier()
    pend = []
    for j in range(NSUB):
        reads[j].wait()
        pend.append(pltpu.async_copy(vb[j], shared.at[is1.at[jnp.int32(j)]],
                                     oss[j & 1], add=True))
    for hh in pend:
        hh.wait()
    plsc.subcore_barrier()

    @pl.when(c == 0)
    def _():
        pltpu.sync_copy(shared.at[pl.ds(base_r, 128)], pb0_h.at[pl.ds(base_r, 128)])

    @pl.when(c == 1)
    def _():
        pltpu.sync_copy(shared.at[pl.ds(base_r, 128)], pb1_h.at[pl.ds(base_r, 128)])


def _mlp_body(h_ref, w1_ref, b1_ref, w2_ref, b2_ref, o_ref):
    t = jnp.dot(h_ref[...], w1_ref[...],
                preferred_element_type=jnp.float32) + b1_ref[...]
    t = jnp.maximum(t, 0.0)
    o_ref[...] = jnp.dot(t, w2_ref[...],
                         preferred_element_type=jnp.float32) + b2_ref[...]


def _fin_body(x_ref, s0_ref, s1_ref, c0_ref, c1_ref, o_ref):
    cnt = c0_ref[...][:, 0:1] + c1_ref[...][:, 0:1]
    ssum = s0_ref[...] + s1_ref[...]
    g = ssum / jnp.maximum(cnt, 1.0)
    o_ref[...] = x_ref[...] + jnp.maximum(g, 0.0)


def kernel(x, edge_index, W1, b1, W2, b2):
    N, D = x.shape
    E = edge_index.shape[1]
    pad = EP - E
    trash = NP - 8

    ei = edge_index.astype(jnp.int32)
    col0, col1 = ei[0], ei[1]
    g0 = jnp.concatenate([col0, jnp.zeros((pad,), jnp.int32)]).reshape(EP // SUB, SUB)
    g1 = jnp.concatenate([col1, jnp.zeros((pad,), jnp.int32)]).reshape(EP // SUB, SUB)
    s1 = jnp.concatenate([col1, jnp.full((pad,), trash, jnp.int32)]).reshape(EP // SUB, SUB)
    b1r = b1.reshape(1, D)
    b2r = b2.reshape(1, D)

    mesh = plsc.VectorSubcoreMesh(core_axis_name="c", subcore_axis_name="s")
    f32 = jnp.float32
    DMA = pltpu.SemaphoreType.DMA

    phase_a = functools.partial(
        pl.kernel, mesh=mesh,
        out_type=(jax.ShapeDtypeStruct((EP, D), f32),
                  jax.ShapeDtypeStruct((NP, D), f32),
                  jax.ShapeDtypeStruct((NP, D), f32),
                  jax.ShapeDtypeStruct((NP, D), f32),
                  jax.ShapeDtypeStruct((NP, D), f32)),
        scratch_types=[pltpu.VMEM((NSUB, SUB), jnp.int32),
                       pltpu.VMEM((NSUB, SUB), jnp.int32),
                       pltpu.VMEM((NSUB, SUB), jnp.int32),
                       pltpu.VMEM((NSUB, SUB), jnp.int32),
                       pltpu.VMEM((SUB, D), f32),
                       pltpu.VMEM((SUB, D), f32),
                       pltpu.VMEM((SUB, D), f32),
                       pltpu.VMEM((SUB, D), f32),
                       pltpu.VMEM((SUB, D), f32),
                       pltpu.VMEM((64, D), f32),
                       pltpu.VMEM_SHARED((NP, D), f32),
                       pltpu.VMEM_SHARED((NP, D), f32),
                       DMA, DMA, DMA, DMA, DMA, DMA, DMA, DMA, DMA, DMA],
    )(_phase_a)
    lgx, ps0, ps1, pc0, pc1 = phase_a(g0, g1, s1, x)

    phase_b = functools.partial(
        pl.kernel, mesh=mesh,
        out_type=jax.ShapeDtypeStruct((EP, D), f32),
        scratch_types=[pltpu.VMEM((NSUB, SUB), jnp.int32),
                       pltpu.VMEM((SUB, D), f32),
                       pltpu.VMEM((SUB, D), f32),
                       pltpu.VMEM((SUB, D), f32),
                       pltpu.VMEM((SUB, D), f32),
                       pltpu.VMEM((SUB, D), f32),
                       pltpu.VMEM((SUB, D), f32),
                       DMA, DMA, DMA, DMA, DMA, DMA, DMA, DMA],
    )(_phase_b)
    h = phase_b(lgx, ps0, ps1, g0)

    BLK = 512
    lgx2 = pl.pallas_call(
        _mlp_body,
        grid=(EP // BLK,),
        in_specs=[pl.BlockSpec((BLK, D), lambda i: (i, _i0)),
                  pl.BlockSpec((D, D), lambda i: (_i0, _i0)),
                  pl.BlockSpec((1, D), lambda i: (_i0, _i0)),
                  pl.BlockSpec((D, D), lambda i: (_i0, _i0)),
                  pl.BlockSpec((1, D), lambda i: (_i0, _i0))],
        out_specs=pl.BlockSpec((BLK, D), lambda i: (i, _i0)),
        out_shape=jax.ShapeDtypeStruct((EP, D), f32),
    )(h, W1, b1r, W2, b2r)

    phase_d = functools.partial(
        pl.kernel, mesh=mesh,
        out_type=(jax.ShapeDtypeStruct((NP, D), f32),
                  jax.ShapeDtypeStruct((NP, D), f32)),
        scratch_types=[pltpu.VMEM((NSUB, SUB), jnp.int32),
                       pltpu.VMEM((SUB, D), f32),
                       pltpu.VMEM((SUB, D), f32),
                       pltpu.VMEM((SUB, D), f32),
                       pltpu.VMEM((SUB, D), f32),
                       pltpu.VMEM((64, D), f32),
                       pltpu.VMEM_SHARED((NP, D), f32),
                       DMA, DMA, DMA, DMA, DMA, DMA],
    )(_phase_d)
    pb0, pb1 = phase_d(lgx2, s1)

    RB = 400
    out = pl.pallas_call(
        _fin_body,
        grid=(N // RB,),
        in_specs=[pl.BlockSpec((RB, D), lambda i: (i, _i0)),
                  pl.BlockSpec((RB, D), lambda i: (i, _i0)),
                  pl.BlockSpec((RB, D), lambda i: (i, _i0)),
                  pl.BlockSpec((RB, D), lambda i: (i, _i0)),
                  pl.BlockSpec((RB, D), lambda i: (i, _i0))],
        out_specs=pl.BlockSpec((RB, D), lambda i: (i, _i0)),
        out_shape=jax.ShapeDtypeStruct((N, D), f32),
    )(x, pb0, pb1, pc0, pc1)
    return out


# speedup vs baseline: 3.1528x; 1.0205x over previous
import functools

import numpy as np

import jax
import jax.numpy as jnp
from jax import lax
from jax.experimental import pallas as pl
from jax.experimental.pallas import tpu as pltpu
from jax.experimental.pallas import tpu_sc as plsc

_i0 = np.int32(0)
NC = 2
NS = 16
L = 16
NW = NC * NS

SUB = 80
NSUB = 4
CPT = SUB * NSUB
EP = CPT * NW
NP = 2048


def _phase_a(g0_h, g1_h, s1_h, x_h,
             lgx_h, ps0_h, ps1_h, pc0_h, pc1_h,
             ig0, ig1, sm, is1,
             a0, b0, a1, b1, a2, b2, ones, zbuf,
             msum, mcnt,
             ga0, ga1, ga2, gb0, gb1, gb2, om, oc, ol):
    c = lax.axis_index("c")
    s = lax.axis_index("s")
    wid = c * NS + s
    D = x_h.shape[1]
    z16 = jnp.zeros((L,), jnp.float32)
    one16 = jnp.ones((L,), jnp.float32)

    rb = wid * NSUB
    pltpu.sync_copy(g0_h.at[pl.ds(rb, NSUB)], ig0)
    pltpu.sync_copy(g1_h.at[pl.ds(rb, NSUB)], ig1)
    pltpu.sync_copy(s1_h.at[pl.ds(rb, NSUB)], is1)

    ab = (a0, a1, a2)
    bb = (b0, b1, b2)
    gsa = (ga0, ga1, ga2)
    gsb = (gb0, gb1, gb2)
    gh = {}
    for j in range(3):
        ha = pltpu.async_copy(x_h.at[ig0.at[jnp.int32(j)]], ab[j], gsa[j])
        hb = pltpu.async_copy(x_h.at[ig1.at[jnp.int32(j)]], bb[j], gsb[j])
        gh[j] = (ha, hb)

    def zrow(i, carry):
        for k in range(D // L):
            zbuf[i, pl.ds(k * L, L)] = z16
        return carry
    lax.fori_loop(0, zbuf.shape[0], zrow, jnp.int32(0))

    def orow(i, carry):
        for k in range(D // L):
            ones[i, pl.ds(k * L, L)] = one16
        return carry
    lax.fori_loop(0, SUB, orow, jnp.int32(0))

    base_r = s * (NP // NS)
    pltpu.sync_copy(zbuf, msum.at[pl.ds(base_r, 64)])
    pltpu.sync_copy(zbuf, msum.at[pl.ds(base_r + 64, 64)])
    pltpu.sync_copy(zbuf, mcnt.at[pl.ds(base_r, 64)])
    pltpu.sync_copy(zbuf, mcnt.at[pl.ds(base_r + 64, 64)])

    for j in range(NSUB):
        for k in range(SUB // L):
            i0 = ig0[j, pl.ds(k * L, L)]
            i1 = ig1[j, pl.ds(k * L, L)]
            sm[j, pl.ds(k * L, L)] = jnp.where(i0 == i1, NP - 8, i1)
    plsc.subcore_barrier()

    pend = {}
    for j in range(NSUB):
        eb = wid * CPT + j * SUB
        if j >= 1 and (j + 2) < NSUB:
            for hh in pend.pop(j - 1):
                hh.wait()
            jj = j + 2
            ha2 = pltpu.async_copy(x_h.at[ig0.at[jnp.int32(jj)]], ab[jj % 3], gsa[jj % 3])
            hb2 = pltpu.async_copy(x_h.at[ig1.at[jnp.int32(jj)]], bb[jj % 3], gsb[jj % 3])
            gh[jj] = (ha2, hb2)
        ha, hb = gh.pop(j)
        ha.wait()
        hb.wait()
        a = ab[j % 3]
        b = bb[j % 3]

        def row_fn(r, carry):
            for k in range(D // L):
                va = a[r, pl.ds(k * L, L)]
                vb = b[r, pl.ds(k * L, L)]
                a[r, pl.ds(k * L, L)] = jnp.maximum(0.5 * va + 1.5 * vb, 0.0)
                b[r, pl.ds(k * L, L)] = 0.5 * va + 0.5 * vb
            return carry
        lax.fori_loop(0, SUB, row_fn, jnp.int32(0))

        pend[j] = (pltpu.async_copy(a, msum.at[sm.at[jnp.int32(j)]], om, add=True),
                   pltpu.async_copy(ones, mcnt.at[is1.at[jnp.int32(j)]], oc, add=True),
                   pltpu.async_copy(b, lgx_h.at[pl.ds(eb, SUB)], ol))

    for j in sorted(pend):
        for hh in pend[j]:
            hh.wait()
    plsc.subcore_barrier()

    @pl.when(c == 0)
    def _():
        pltpu.sync_copy(msum.at[pl.ds(base_r, 128)], ps0_h.at[pl.ds(base_r, 128)])
        pltpu.sync_copy(mcnt.at[pl.ds(base_r, 128)], pc0_h.at[pl.ds(base_r, 128)])

    @pl.when(c == 1)
    def _():
        pltpu.sync_copy(msum.at[pl.ds(base_r, 128)], ps1_h.at[pl.ds(base_r, 128)])
        pltpu.sync_copy(mcnt.at[pl.ds(base_r, 128)], pc1_h.at[pl.ds(base_r, 128)])


def _phase_b(ps0_h, ps1_h, g0_h, agg_h,
             ig0, p00, p10, p01, p11, p02, p12, p03, p13,
             g00, g10, g01, g11, g02, g12, g03, g13, oh):
    c = lax.axis_index("c")
    s = lax.axis_index("s")
    wid = c * NS + s
    D = p00.shape[1]
    rb = wid * NSUB
    pltpu.sync_copy(g0_h.at[pl.ds(rb, NSUB)], ig0)

    p0b = (p00, p01, p02, p03)
    p1b = (p10, p11, p12, p13)
    g0s = (g00, g01, g02, g03)
    g1s = (g10, g11, g12, g13)

    gh = []
    for j in range(NSUB):
        h0 = pltpu.async_copy(ps0_h.at[ig0.at[jnp.int32(j)]], p0b[j], g0s[j])
        h1 = pltpu.async_copy(ps1_h.at[ig0.at[jnp.int32(j)]], p1b[j], g1s[j])
        gh.append((h0, h1))

    pend = []
    for j in range(NSUB):
        eb = wid * CPT + j * SUB
        h0, h1 = gh[j]
        h0.wait()
        h1.wait()
        p0 = p0b[j]
        p1 = p1b[j]

        def row_fn(r, carry):
            for k in range(D // L):
                p0[r, pl.ds(k * L, L)] = (p0[r, pl.ds(k * L, L)]
                                          + p1[r, pl.ds(k * L, L)])
            return carry
        lax.fori_loop(0, SUB, row_fn, jnp.int32(0))
        pend.append(pltpu.async_copy(p0, agg_h.at[pl.ds(eb, SUB)], oh))

    for hh in pend:
        hh.wait()


def _phase_d(lgx2_h, s1_h, pb0_h, pb1_h,
             is1, v0, v1, v2, v3, zbuf, shared,
             gv0, gv1, gv2, gv3, os0, os1):
    c = lax.axis_index("c")
    s = lax.axis_index("s")
    wid = c * NS + s
    D = v0.shape[1]
    z16 = jnp.zeros((L,), jnp.float32)

    def zrow(i, carry):
        for k in range(D // L):
            zbuf[i, pl.ds(k * L, L)] = z16
        return carry
    lax.fori_loop(0, zbuf.shape[0], zrow, jnp.int32(0))

    base_r = s * (NP // NS)
    pltpu.sync_copy(zbuf, shared.at[pl.ds(base_r, 64)])
    pltpu.sync_copy(zbuf, shared.at[pl.ds(base_r + 64, 64)])

    rb = wid * NSUB
    pltpu.sync_copy(s1_h.at[pl.ds(rb, NSUB)], is1)
    vb = (v0, v1, v2, v3)
    gvs = (gv0, gv1, gv2, gv3)
    oss = (os0, os1)
    reads = []
    for j in range(NSUB):
        eb = wid * CPT + j * SUB
        reads.append(pltpu.async_copy(lgx2_h.at[pl.ds(eb, SUB)], vb[j], gvs[j]))
    plsc.subcore_barrier()
    pend = []
    for j in range(NSUB):
        reads[j].wait()
        pend.append(pltpu.async_copy(vb[j], shared.at[is1.at[jnp.int32(j)]],
                                     oss[j & 1], add=True))
    for hh in pend:
        hh.wait()
    plsc.subcore_barrier()

    @pl.when(c == 0)
    def _():
        pltpu.sync_copy(shared.at[pl.ds(base_r, 128)], pb0_h.at[pl.ds(base_r, 128)])

    @pl.when(c == 1)
    def _():
        pltpu.sync_copy(shared.at[pl.ds(base_r, 128)], pb1_h.at[pl.ds(base_r, 128)])


def _mlp_body(lgx_ref, agg_ref, w1_ref, b1_ref, w2_ref, b2_ref, o_ref):
    h = lgx_ref[...] + agg_ref[...]
    t = jnp.dot(h, w1_ref[...],
                preferred_element_type=jnp.float32) + b1_ref[...]
    t = jnp.maximum(t, 0.0)
    o_ref[...] = jnp.dot(t, w2_ref[...],
                         preferred_element_type=jnp.float32) + b2_ref[...]


def _fin_body(x_ref, s0_ref, s1_ref, c0_ref, c1_ref, o_ref):
    cnt = c0_ref[...][:, 0:1] + c1_ref[...][:, 0:1]
    ssum = s0_ref[...] + s1_ref[...]
    g = ssum / jnp.maximum(cnt, 1.0)
    o_ref[...] = x_ref[...] + jnp.maximum(g, 0.0)


def kernel(x, edge_index, W1, b1, W2, b2):
    N, D = x.shape
    E = edge_index.shape[1]
    pad = EP - E
    trash = NP - 8

    ei = edge_index.astype(jnp.int32)
    col0, col1 = ei[0], ei[1]
    g0 = jnp.concatenate([col0, jnp.zeros((pad,), jnp.int32)]).reshape(EP // SUB, SUB)
    g1 = jnp.concatenate([col1, jnp.zeros((pad,), jnp.int32)]).reshape(EP // SUB, SUB)
    s1 = jnp.concatenate([col1, jnp.full((pad,), trash, jnp.int32)]).reshape(EP // SUB, SUB)
    b1r = b1.reshape(1, D)
    b2r = b2.reshape(1, D)

    mesh = plsc.VectorSubcoreMesh(core_axis_name="c", subcore_axis_name="s")
    f32 = jnp.float32
    DMA = pltpu.SemaphoreType.DMA

    phase_a = functools.partial(
        pl.kernel, mesh=mesh,
        out_type=(jax.ShapeDtypeStruct((EP, D), f32),
                  jax.ShapeDtypeStruct((NP, D), f32),
                  jax.ShapeDtypeStruct((NP, D), f32),
                  jax.ShapeDtypeStruct((NP, D), f32),
                  jax.ShapeDtypeStruct((NP, D), f32)),
        scratch_types=[pltpu.VMEM((NSUB, SUB), jnp.int32),
                       pltpu.VMEM((NSUB, SUB), jnp.int32),
                       pltpu.VMEM((NSUB, SUB), jnp.int32),
                       pltpu.VMEM((NSUB, SUB), jnp.int32)]
                      + [pltpu.VMEM((SUB, D), f32)] * 7
                      + [pltpu.VMEM((64, D), f32),
                         pltpu.VMEM_SHARED((NP, D), f32),
                         pltpu.VMEM_SHARED((NP, D), f32)]
                      + [DMA] * 9,
    )(_phase_a)
    lgx, ps0, ps1, pc0, pc1 = phase_a(g0, g1, s1, x)

    phase_b = functools.partial(
        pl.kernel, mesh=mesh,
        out_type=jax.ShapeDtypeStruct((EP, D), f32),
        scratch_types=[pltpu.VMEM((NSUB, SUB), jnp.int32)]
                      + [pltpu.VMEM((SUB, D), f32)] * 8
                      + [DMA] * 9,
    )(_phase_b)
    agg = phase_b(ps0, ps1, g0)

    BLK = 512
    lgx2 = pl.pallas_call(
        _mlp_body,
        grid=(EP // BLK,),
        in_specs=[pl.BlockSpec((BLK, D), lambda i: (i, _i0)),
                  pl.BlockSpec((BLK, D), lambda i: (i, _i0)),
                  pl.BlockSpec((D, D), lambda i: (_i0, _i0)),
                  pl.BlockSpec((1, D), lambda i: (_i0, _i0)),
                  pl.BlockSpec((D, D), lambda i: (_i0, _i0)),
                  pl.BlockSpec((1, D), lambda i: (_i0, _i0))],
        out_specs=pl.BlockSpec((BLK, D), lambda i: (i, _i0)),
        out_shape=jax.ShapeDtypeStruct((EP, D), f32),
    )(lgx, agg, W1, b1r, W2, b2r)

    phase_d = functools.partial(
        pl.kernel, mesh=mesh,
        out_type=(jax.ShapeDtypeStruct((NP, D), f32),
                  jax.ShapeDtypeStruct((NP, D), f32)),
        scratch_types=[pltpu.VMEM((NSUB, SUB), jnp.int32),
                       pltpu.VMEM((SUB, D), f32),
                       pltpu.VMEM((SUB, D), f32),
                       pltpu.VMEM((SUB, D), f32),
                       pltpu.VMEM((SUB, D), f32),
                       pltpu.VMEM((64, D), f32),
                       pltpu.VMEM_SHARED((NP, D), f32),
                       DMA, DMA, DMA, DMA, DMA, DMA],
    )(_phase_d)
    pb0, pb1 = phase_d(lgx2, s1)

    RB = 400
    out = pl.pallas_call(
        _fin_body,
        grid=(N // RB,),
        in_specs=[pl.BlockSpec((RB, D), lambda i: (i, _i0)),
                  pl.BlockSpec((RB, D), lambda i: (i, _i0)),
                  pl.BlockSpec((RB, D), lambda i: (i, _i0)),
                  pl.BlockSpec((RB, D), lambda i: (i, _i0)),
                  pl.BlockSpec((RB, D), lambda i: (i, _i0))],
        out_specs=pl.BlockSpec((RB, D), lambda i: (i, _i0)),
        out_shape=jax.ShapeDtypeStruct((N, D), f32),
    )(x, pb0, pb1, pc0, pc1)
    return out
